# baseline (speedup 1.0000x reference)
import jax
import jax.numpy as jnp
from jax import lax
from jax.experimental import pallas as pl
from jax.experimental.pallas import tpu as pltpu
from jax.experimental.pallas import tpu_sc as plsc

_N = 1000000
_D = 64
_B = 16384
_NC = 2
_NS = 16
_NW = _NC * _NS
_BPW = _B // _NW
_CHUNK = 128
_NCHUNK = _BPW // _CHUNK


def _gather_body(n_id_hbm, mem_t_hbm, last_hbm, out_t_hbm, last_out_hbm,
                 idx_v, stage_v, last_v, sem_rows, sem_last):
    wid = lax.axis_index("s") * _NC + lax.axis_index("c")
    base = wid * _BPW

    pltpu.sync_copy(n_id_hbm.at[wid], idx_v)

    last_copies = [
        pltpu.async_copy(last_hbm.at[idx_v.at[c]],
                         last_v.at[pl.ds(c * _CHUNK, _CHUNK)], sem_last)
        for c in range(_NCHUNK)
    ]

    def jbody(j, carry):
        for c in range(_NCHUNK):
            pltpu.async_copy(mem_t_hbm.at[j].at[idx_v.at[c]],
                             stage_v.at[j, pl.ds(c * _CHUNK, _CHUNK)],
                             sem_rows)
        return carry

    lax.fori_loop(0, _D, jbody, 0, unroll=False)

    pltpu.make_async_copy(
        out_t_hbm.at[:, pl.ds(base, _BPW)], stage_v, sem_rows).wait()

    pltpu.sync_copy(stage_v, out_t_hbm.at[:, pl.ds(base, _BPW)])

    for c in last_copies:
        c.wait()
    pltpu.sync_copy(last_v, last_out_hbm.at[pl.ds(base, _BPW)])


_gather_call = pl.kernel(
    _gather_body,
    out_type=(
        jax.ShapeDtypeStruct((_D, _B), jnp.float32),
        jax.ShapeDtypeStruct((_B,), jnp.int32),
    ),
    mesh=plsc.VectorSubcoreMesh(
        core_axis_name="c", subcore_axis_name="s",
        num_cores=_NC, num_subcores=_NS),
    scratch_types=[
        pltpu.VMEM((_NCHUNK, _CHUNK), jnp.int32),
        pltpu.VMEM((_D, _BPW), jnp.float32),
        pltpu.VMEM((_BPW,), jnp.int32),
        pltpu.SemaphoreType.DMA,
        pltpu.SemaphoreType.DMA,
    ],
    compiler_params=pltpu.CompilerParams(use_tc_tiling_on_sc=False),
)


@jax.jit
def kernel(n_id, memory, last_update):
    n_id_r = n_id.reshape(_NW, _NCHUNK, _CHUNK)
    out_t, last_out = _gather_call(n_id_r, memory.T, last_update)
    return (out_t.T, last_out)


# speedup vs baseline: 8.0357x; 8.0357x over previous
import jax
import jax.numpy as jnp
from jax import lax
from jax.experimental import pallas as pl
from jax.experimental.pallas import tpu as pltpu
from jax.experimental.pallas import tpu_sc as plsc

_D = 64
_B = 16384
_NC = 2
_NS = 16
_NW = _NC * _NS
_BPW = _B // _NW
_CHUNK = 128
_NCHUNK = _BPW // _CHUNK


def _gather_body(n_id_hbm, memory_hbm, last_hbm, mem_out_hbm, last_out_hbm,
                 idx_v, rows_v, last_v, sem_rows, sem_last):
    wid = lax.axis_index("s") * _NC + lax.axis_index("c")
    base = wid * _BPW

    pltpu.sync_copy(n_id_hbm.at[wid], idx_v)

    copies = []
    for j in range(_NCHUNK):
        idx_j = idx_v.at[j]
        copies.append(pltpu.async_copy(
            memory_hbm.at[idx_j], rows_v.at[pl.ds(j * _CHUNK, _CHUNK)],
            sem_rows))
        copies.append(pltpu.async_copy(
            last_hbm.at[idx_j], last_v.at[pl.ds(j * _CHUNK, _CHUNK)],
            sem_last))
    for c in copies:
        c.wait()

    pltpu.sync_copy(rows_v, mem_out_hbm.at[pl.ds(base, _BPW)])
    pltpu.sync_copy(last_v, last_out_hbm.at[pl.ds(base, _BPW)])


_gather_call = pl.kernel(
    _gather_body,
    out_type=(
        jax.ShapeDtypeStruct((_B, _D), jnp.float32),
        jax.ShapeDtypeStruct((_B,), jnp.int32),
    ),
    mesh=plsc.VectorSubcoreMesh(
        core_axis_name="c", subcore_axis_name="s",
        num_cores=_NC, num_subcores=_NS),
    scratch_types=[
        pltpu.VMEM((_NCHUNK, _CHUNK), jnp.int32),
        pltpu.VMEM((_BPW, _D), jnp.float32),
        pltpu.VMEM((_BPW,), jnp.int32),
        pltpu.SemaphoreType.DMA,
        pltpu.SemaphoreType.DMA,
    ],
    compiler_params=pltpu.CompilerParams(use_tc_tiling_on_sc=False),
)


@jax.jit
def kernel(n_id, memory, last_update):
    n_id_r = n_id.reshape(_NW, _NCHUNK, _CHUNK)
    mem_out, last_out = _gather_call(n_id_r, memory, last_update)
    return (mem_out, last_out)
